# baseline (speedup 1.0000x reference)
import functools

import jax
import jax.numpy as jnp
from jax import lax
from jax.experimental import pallas as pl
from jax.experimental.pallas import tpu as pltpu
from jax.experimental.pallas import tpu_sc as plsc

HIDDEN = 128
SEQ = 200
EPS = 1e-12
L = 16
NVH = HIDDEN // L
NC = 2
NS = 16
NW = NC * NS
CHUNK = 128
NBUF = 2


def _xlane_sum(v, rots):
    for idx in rots:
        v = v + v.at[idx].get(mode="promise_in_bounds")
    return v


def _rsqrt(x):
    i = plsc.bitcast(x, jnp.int32)
    i = jnp.int32(0x5F3759DF) - lax.shift_right_logical(i, 1)
    y = plsc.bitcast(i, jnp.float32)
    for _ in range(2):
        y = y * (1.5 - 0.5 * x * y * y)
    return y


def _tree_sum(vs):
    vs = list(vs)
    while len(vs) > 1:
        nxt = [vs[i] + vs[i + 1] for i in range(0, len(vs) - 1, 2)]
        if len(vs) % 2:
            nxt.append(vs[-1])
        vs = nxt
    return vs[0]


def _body(ids_hbm, comb_hbm, word_hbm, pt_hbm, gb_hbm, out_hbm,
          ids_v, cbuf_v, pt_v, gb_v, wrows_v, orows_v,
          st_v, sq_v, acc_v, gsems, csems, osems):
    n_chunks = ids_hbm.shape[1]
    tpw = n_chunks * CHUNK

    wid = lax.axis_index("s") * NC + lax.axis_index("c")
    base = wid * tpw

    pltpu.sync_copy(ids_hbm.at[wid], ids_v)
    pltpu.sync_copy(pt_hbm, pt_v)
    pltpu.sync_copy(gb_hbm, gb_v)

    gvecs = [gb_v[0, pl.ds(h * L, L)] for h in range(NVH)]
    bvecs = [gb_v[1, pl.ds(h * L, L)] for h in range(NVH)]
    lanes = lax.iota(jnp.int32, L)
    rots = [lax.rem(lanes + sh, jnp.int32(L)) for sh in (8, 4, 2, 1)]

    def start_gather(c, b):
        pltpu.async_copy(
            word_hbm.at[ids_v.at[c]], wrows_v.at[b], gsems.at[b])
        pltpu.async_copy(
            comb_hbm.at[wid, c], cbuf_v.at[b], csems.at[b])

    def wait_gather(b):
        pltpu.make_async_copy(
            word_hbm.at[ids_v.at[0]],
            wrows_v.at[b], gsems.at[b]).wait()
        pltpu.make_async_copy(
            comb_hbm.at[wid, 0], cbuf_v.at[b], csems.at[b]).wait()

    def wait_out(b):
        pltpu.make_async_copy(
            orows_v.at[b], out_hbm.at[pl.ds(base, CHUNK)],
            osems.at[b]).wait()

    for b in range(NBUF):
        start_gather(b, b)

    def run_groups(b):
        def group_body(g, carry):
            cvec = cbuf_v[b, pl.ds(g * L, L)]
            for lane in range(L):
                j = g * L + lane
                cj = cvec[lane]
                acc = [wrows_v[b, j, pl.ds(h * L, L)]
                       + pt_v[cj, pl.ds(h * L, L)]
                       for h in range(NVH)]
                for h in range(NVH):
                    acc_v[lane, pl.ds(h * L, L)] = acc[h]
                st_v[lane, :] = _tree_sum(acc)
                sq_v[lane, :] = _tree_sum([a * a for a in acc])
            tot = _tree_sum([
                plsc.load_gather(st_v, [lanes, jnp.full((L,), k, jnp.int32)])
                for k in range(L)])
            totq = _tree_sum([
                plsc.load_gather(sq_v, [lanes, jnp.full((L,), k, jnp.int32)])
                for k in range(L)])
            u_vec = tot * (1.0 / HIDDEN)
            var_vec = totq * (1.0 / HIDDEN) - u_vec * u_vec
            inv_vec = _rsqrt(var_vec + EPS)
            for lane in range(L):
                j = g * L + lane
                lane_idx = jnp.full((L,), lane, jnp.int32)
                u_b = u_vec.at[lane_idx].get(mode="promise_in_bounds")
                inv_b = inv_vec.at[lane_idx].get(mode="promise_in_bounds")
                for h in range(NVH):
                    a = acc_v[lane, pl.ds(h * L, L)]
                    orows_v[b, j, pl.ds(h * L, L)] = (
                        (a - u_b) * inv_b * gvecs[h] + bvecs[h])
            return carry

        lax.fori_loop(0, CHUNK // L, group_body, 0)

    def do_chunk(c, b, last):
        wait_gather(b)

        @pl.when(c >= NBUF)
        def _():
            wait_out(b)

        run_groups(b)

        if not last:
            nxt = c + NBUF

            @pl.when(nxt < n_chunks)
            def _():
                start_gather(nxt, b)

        pltpu.async_copy(
            orows_v.at[b],
            out_hbm.at[pl.ds(base + c * CHUNK, CHUNK)],
            osems.at[b])

    def chunk_round(g, _):
        for b in range(NBUF):
            do_chunk(g * NBUF + b, b, False)
        return 0

    lax.fori_loop(0, n_chunks // NBUF, chunk_round, 0)
    for c in range(n_chunks - n_chunks % NBUF, n_chunks):
        do_chunk(c, c % NBUF, True)
    for b in range(NBUF):
        wait_out(b)


def kernel(input_ids, token_type_ids, word_emb, pos_emb, type_emb,
           gamma, beta):
    batch, seq = input_ids.shape
    n_tok = batch * seq
    n_chunks_pw = n_tok // (NW * CHUNK)

    ids_3d = input_ids.reshape(NW, n_chunks_pw, CHUNK).astype(jnp.int32)
    pos_row = jnp.arange(seq, dtype=jnp.int32)
    comb = (token_type_ids.astype(jnp.int32) * seq
            + pos_row[None, :]).reshape(NW, n_chunks_pw, CHUNK)
    pt = (pos_emb[:seq][None, :, :] + type_emb[:, None, :]).reshape(
        type_emb.shape[0] * seq, HIDDEN)
    gb = jnp.stack([gamma, beta])

    mesh = plsc.VectorSubcoreMesh(core_axis_name="c", subcore_axis_name="s",
                                  num_cores=NC, num_subcores=NS)
    run = pl.kernel(
        _body,
        out_type=jax.ShapeDtypeStruct((n_tok, HIDDEN), jnp.float32),
        mesh=mesh,
        compiler_params=pltpu.CompilerParams(needs_layout_passes=False),
        scratch_types=[
            pltpu.VMEM((n_chunks_pw, CHUNK), jnp.int32),
            pltpu.VMEM((NBUF, CHUNK), jnp.int32),
            pltpu.VMEM((pt.shape[0], HIDDEN), jnp.float32),
            pltpu.VMEM((2, HIDDEN), jnp.float32),
            pltpu.VMEM((NBUF, CHUNK, HIDDEN), jnp.float32),
            pltpu.VMEM((NBUF, CHUNK, HIDDEN), jnp.float32),
            pltpu.VMEM((L, L), jnp.float32),
            pltpu.VMEM((L, L), jnp.float32),
            pltpu.VMEM((L, HIDDEN), jnp.float32),
            pltpu.SemaphoreType.DMA((NBUF,)),
            pltpu.SemaphoreType.DMA((NBUF,)),
            pltpu.SemaphoreType.DMA((NBUF,)),
        ],
    )
    out = run(ids_3d, comb, word_emb, pt, gb)
    return out.reshape(batch, seq, HIDDEN)


# speedup vs baseline: 2.3161x; 2.3161x over previous
import functools

import jax
import jax.numpy as jnp
from jax import lax
from jax.experimental import pallas as pl
from jax.experimental.pallas import tpu as pltpu
from jax.experimental.pallas import tpu_sc as plsc

HIDDEN = 128
SEQ = 200
EPS = 1e-12
L = 16
NVH = HIDDEN // L
NC = 2
NS = 16
NW = NC * NS
CHUNK = 128
NBUF = 2


def _xlane_sum(v, rots):
    for idx in rots:
        v = v + v.at[idx].get(mode="promise_in_bounds")
    return v


def _rsqrt(x):
    i = plsc.bitcast(x, jnp.int32)
    i = jnp.int32(0x5F3759DF) - lax.shift_right_logical(i, 1)
    y = plsc.bitcast(i, jnp.float32)
    for _ in range(1):
        y = y * (1.5 - 0.5 * x * y * y)
    return y


def _tree_sum(vs):
    vs = list(vs)
    while len(vs) > 1:
        nxt = [vs[i] + vs[i + 1] for i in range(0, len(vs) - 1, 2)]
        if len(vs) % 2:
            nxt.append(vs[-1])
        vs = nxt
    return vs[0]


def _body(ids_hbm, comb_hbm, word_hbm, pt_hbm, gb_hbm, out_hbm,
          ids_v, cbuf_v, pt_v, gb_v, wrows_v, orows_v,
          gsems, csems, osems):
    n_chunks = ids_hbm.shape[1]
    tpw = n_chunks * CHUNK

    wid = lax.axis_index("s") * NC + lax.axis_index("c")
    base = wid * tpw

    pltpu.sync_copy(ids_hbm.at[wid], ids_v)
    pltpu.sync_copy(pt_hbm, pt_v)
    pltpu.sync_copy(gb_hbm, gb_v)

    gvecs = [gb_v[0, pl.ds(h * L, L)] for h in range(NVH)]
    bvecs = [gb_v[1, pl.ds(h * L, L)] for h in range(NVH)]
    lanes = lax.iota(jnp.int32, L)
    rots = [lax.rem(lanes + sh, jnp.int32(L)) for sh in (8, 4, 2, 1)]

    def start_gather(c, b):
        pltpu.async_copy(
            word_hbm.at[ids_v.at[c]], wrows_v.at[b], gsems.at[b])
        pltpu.async_copy(
            comb_hbm.at[wid, c], cbuf_v.at[b], csems.at[b])

    def wait_gather(b):
        pltpu.make_async_copy(
            word_hbm.at[ids_v.at[0]],
            wrows_v.at[b], gsems.at[b]).wait()
        pltpu.make_async_copy(
            comb_hbm.at[wid, 0], cbuf_v.at[b], csems.at[b]).wait()

    def wait_out(b):
        pltpu.make_async_copy(
            orows_v.at[b], out_hbm.at[pl.ds(base, CHUNK)],
            osems.at[b]).wait()

    for b in range(NBUF):
        start_gather(b, b)

    def run_groups(b):
        def group_body(g, carry):
            cvec = cbuf_v[b, pl.ds(g * L, L)]
            for lane in range(L):
                j = g * L + lane
                cj = cvec[lane]
                acc = [wrows_v[b, j, pl.ds(h * L, L)]
                       + pt_v[cj, pl.ds(h * L, L)]
                       for h in range(NVH)]
                ssum = _tree_sum(acc)
                ssq = _tree_sum([a * a for a in acc])
                u_b = _xlane_sum(ssum, rots) * (1.0 / HIDDEN)
                var = _xlane_sum(ssq, rots) * (1.0 / HIDDEN) - u_b * u_b
                inv_b = _rsqrt(var + EPS)
                for h in range(NVH):
                    orows_v[b, j, pl.ds(h * L, L)] = (
                        (acc[h] - u_b) * inv_b * gvecs[h] + bvecs[h])
            return carry

        lax.fori_loop(0, CHUNK // L, group_body, 0)

    def do_chunk(c, b, last):
        wait_gather(b)

        @pl.when(c >= NBUF)
        def _():
            wait_out(b)

        run_groups(b)

        if not last:
            nxt = c + NBUF

            @pl.when(nxt < n_chunks)
            def _():
                start_gather(nxt, b)

        pltpu.async_copy(
            orows_v.at[b],
            out_hbm.at[pl.ds(base + c * CHUNK, CHUNK)],
            osems.at[b])

    def chunk_round(g, _):
        for b in range(NBUF):
            do_chunk(g * NBUF + b, b, False)
        return 0

    lax.fori_loop(0, n_chunks // NBUF, chunk_round, 0)
    for c in range(n_chunks - n_chunks % NBUF, n_chunks):
        do_chunk(c, c % NBUF, True)
    for b in range(NBUF):
        wait_out(b)


def kernel(input_ids, token_type_ids, word_emb, pos_emb, type_emb,
           gamma, beta):
    batch, seq = input_ids.shape
    n_tok = batch * seq
    n_chunks_pw = n_tok // (NW * CHUNK)

    ids_3d = input_ids.reshape(NW, n_chunks_pw, CHUNK).astype(jnp.int32)
    pos_row = jnp.arange(seq, dtype=jnp.int32)
    comb = (token_type_ids.astype(jnp.int32) * seq
            + pos_row[None, :]).reshape(NW, n_chunks_pw, CHUNK)
    pt = (pos_emb[:seq][None, :, :] + type_emb[:, None, :]).reshape(
        type_emb.shape[0] * seq, HIDDEN)
    gb = jnp.stack([gamma, beta])

    mesh = plsc.VectorSubcoreMesh(core_axis_name="c", subcore_axis_name="s",
                                  num_cores=NC, num_subcores=NS)
    run = pl.kernel(
        _body,
        out_type=jax.ShapeDtypeStruct((n_tok, HIDDEN), jnp.float32),
        mesh=mesh,
        compiler_params=pltpu.CompilerParams(needs_layout_passes=False),
        scratch_types=[
            pltpu.VMEM((n_chunks_pw, CHUNK), jnp.int32),
            pltpu.VMEM((NBUF, CHUNK), jnp.int32),
            pltpu.VMEM((pt.shape[0], HIDDEN), jnp.float32),
            pltpu.VMEM((2, HIDDEN), jnp.float32),
            pltpu.VMEM((NBUF, CHUNK, HIDDEN), jnp.float32),
            pltpu.VMEM((NBUF, CHUNK, HIDDEN), jnp.float32),
            pltpu.SemaphoreType.DMA((NBUF,)),
            pltpu.SemaphoreType.DMA((NBUF,)),
            pltpu.SemaphoreType.DMA((NBUF,)),
        ],
    )
    out = run(ids_3d, comb, word_emb, pt, gb)
    return out.reshape(batch, seq, HIDDEN)
